# baseline (speedup 1.0000x reference)
import functools

import jax
import jax.numpy as jnp
from jax import lax
from jax.experimental import pallas as pl
from jax.experimental.pallas import tpu as pltpu
from jax.experimental.pallas import tpu_sc as plsc

DIM = 16
L = 16
NC, NS = 2, 16
NW = NC * NS
CH = 128


def _sc_products(u, i, j, W, H):
    B = u.shape[0]
    bpw = B // NW
    nch = bpw // CH

    u3 = u.reshape(NW, nch, CH)
    i3 = i.reshape(NW, nch, CH)
    j3 = j.reshape(NW, nch, CH)

    mesh = plsc.VectorSubcoreMesh(
        core_axis_name="c", subcore_axis_name="s",
        num_cores=NC, num_subcores=NS)

    @functools.partial(
        pl.kernel,
        out_type=jax.ShapeDtypeStruct((NW, bpw, DIM), jnp.float32),
        mesh=mesh,
        scratch_types=[
            pltpu.VMEM((nch, CH), jnp.int32),
            pltpu.VMEM((nch, CH), jnp.int32),
            pltpu.VMEM((nch, CH), jnp.int32),
            pltpu.VMEM((bpw, DIM), jnp.float32),
            pltpu.VMEM((bpw, DIM), jnp.float32),
            pltpu.VMEM((bpw, DIM), jnp.float32),
            pltpu.SemaphoreType.DMA,
        ],
        compiler_params=pltpu.CompilerParams(use_tc_tiling_on_sc=False),
    )
    def sc(u_hbm, i_hbm, j_hbm, w_hbm, h_hbm, out_hbm,
           u_v, i_v, j_v, wu_v, hi_v, hj_v, sem):
        wid = lax.axis_index("s") * NC + lax.axis_index("c")
        pltpu.sync_copy(u_hbm.at[wid], u_v)
        pltpu.sync_copy(i_hbm.at[wid], i_v)
        pltpu.sync_copy(j_hbm.at[wid], j_v)
        copies = []
        for k in range(nch):
            dst = pl.ds(k * CH, CH)
            copies.append(pltpu.async_copy(w_hbm.at[u_v.at[k]], wu_v.at[dst], sem))
            copies.append(pltpu.async_copy(h_hbm.at[i_v.at[k]], hi_v.at[dst], sem))
            copies.append(pltpu.async_copy(h_hbm.at[j_v.at[k]], hj_v.at[dst], sem))
        for cp in copies:
            cp.wait()

        def row(r, carry):
            wu = wu_v[r, :]
            hi = hi_v[r, :]
            hj = hj_v[r, :]
            hj_v[r, :] = wu * (hi - hj)
            return carry

        lax.fori_loop(0, bpw, row, 0)
        pltpu.sync_copy(hj_v, out_hbm.at[wid])

    return sc(u3, i3, j3, W, H).reshape(B, DIM)


def _tc_loss(p):
    B = p.shape[0]

    def body(p_ref, o_ref):
        x = jnp.sum(p_ref[...], axis=1)
        t = -x
        sp = jnp.maximum(t, 0.0) + jnp.log1p(jnp.exp(-jnp.abs(t)))
        o_ref[0, 0] = jnp.sum(sp) * (1.0 / B)

    out = pl.pallas_call(
        body,
        out_shape=jax.ShapeDtypeStruct((1, 1), jnp.float32),
        out_specs=pl.BlockSpec(memory_space=pltpu.SMEM),
    )(p)
    return out[0, 0]


def kernel(u, i, j, W, H):
    p = _sc_products(u, i, j, W, H)
    return _tc_loss(p)


# speedup vs baseline: 4.5395x; 4.5395x over previous
import functools

import jax
import jax.numpy as jnp
from jax import lax
from jax.experimental import pallas as pl
from jax.experimental.pallas import tpu as pltpu
from jax.experimental.pallas import tpu_sc as plsc

DIM = 16
L = 16
NC, NS = 2, 16
NW = NC * NS
GRP = 16


def _sc_scores(u, i, j, W, H):
    B = u.shape[0]
    bpw = B // NW

    u2 = u.reshape(NW, bpw)
    i2 = i.reshape(NW, bpw)
    j2 = j.reshape(NW, bpw)
    WT = jnp.transpose(W)
    HT = jnp.transpose(H)

    mesh = plsc.VectorSubcoreMesh(
        core_axis_name="c", subcore_axis_name="s",
        num_cores=NC, num_subcores=NS)

    @functools.partial(
        pl.kernel,
        out_type=jax.ShapeDtypeStruct((NW, bpw), jnp.float32),
        mesh=mesh,
        scratch_types=[
            pltpu.VMEM((bpw,), jnp.int32),
            pltpu.VMEM((bpw,), jnp.int32),
            pltpu.VMEM((bpw,), jnp.int32),
            pltpu.VMEM((GRP * DIM, 128), jnp.float32),
            pltpu.VMEM((GRP * DIM, 128), jnp.float32),
            pltpu.VMEM((GRP * DIM, 128), jnp.float32),
            pltpu.VMEM((bpw,), jnp.float32),
            pltpu.SemaphoreType.DMA,
        ],
        compiler_params=pltpu.CompilerParams(
            use_tc_tiling_on_sc=True, needs_layout_passes=False),
    )
    def sc(u_hbm, i_hbm, j_hbm, wt_hbm, ht_hbm, out_hbm,
           u_v, i_v, j_v, ws_v, his_v, hjs_v, x_v, sem):
        wid = lax.axis_index("s") * NC + lax.axis_index("c")
        pltpu.sync_copy(u_hbm.at[wid], u_v)
        pltpu.sync_copy(i_hbm.at[wid], i_v)
        pltpu.sync_copy(j_hbm.at[wid], j_v)

        lane = lax.iota(jnp.int32, L)

        def fetch(g, carry):
            sl = pl.ds(g * GRP, GRP)
            uvec = u_v[sl]
            ivec = i_v[sl]
            jvec = j_v[sl]
            ub = (uvec >> 7) * 128
            ib = (ivec >> 7) * 128
            jb = (jvec >> 7) * 128
            copies = []
            for t in range(GRP):
                dst = pl.ds(t * DIM, DIM)
                ru = pl.multiple_of(ub[t], 128)
                ri = pl.multiple_of(ib[t], 128)
                rj = pl.multiple_of(jb[t], 128)
                copies.append(pltpu.async_copy(
                    wt_hbm.at[:, pl.ds(ru, 128)], ws_v.at[dst], sem))
                copies.append(pltpu.async_copy(
                    ht_hbm.at[:, pl.ds(ri, 128)], his_v.at[dst], sem))
                copies.append(pltpu.async_copy(
                    ht_hbm.at[:, pl.ds(rj, 128)], hjs_v.at[dst], sem))
            for cp in copies:
                cp.wait()

            cu = uvec & 127
            ci = ivec & 127
            cj = jvec & 127
            rowbase = lane * DIM
            acc = jnp.zeros((L,), jnp.float32)
            for d in range(DIM):
                rf = rowbase + d
                wu = plsc.load_gather(ws_v, [rf, cu])
                hi = plsc.load_gather(his_v, [rf, ci])
                hj = plsc.load_gather(hjs_v, [rf, cj])
                acc = acc + wu * (hi - hj)
            x_v[sl] = acc
            return carry

        lax.fori_loop(0, bpw // GRP, fetch, 0)
        pltpu.sync_copy(x_v, out_hbm.at[wid])

    return sc(u2, i2, j2, WT, HT).reshape(B)


def _tc_loss(x):
    B = x.shape[0]
    xm = x.reshape(B // 128, 128)

    def body(x_ref, o_ref):
        t = -x_ref[...]
        sp = jnp.maximum(t, 0.0) + jnp.log1p(jnp.exp(-jnp.abs(t)))
        o_ref[0, 0] = jnp.sum(sp) * (1.0 / B)

    out = pl.pallas_call(
        body,
        out_shape=jax.ShapeDtypeStruct((1, 1), jnp.float32),
        out_specs=pl.BlockSpec(memory_space=pltpu.SMEM),
    )(xm)
    return out[0, 0]


def kernel(u, i, j, W, H):
    x = _sc_scores(u, i, j, W, H)
    return _tc_loss(x)
